# baseline (speedup 1.0000x reference)
import jax
import jax.numpy as jnp
from jax.experimental import pallas as pl
from jax.experimental.pallas import tpu as pltpu

_BBLK = 8
_K = 64


def _topk_bitonic(scores, pump=None):
    R, G = scores.shape
    lane_g = jax.lax.broadcasted_iota(jnp.int32, (R, G), 1)
    v = scores
    ix = lane_g

    def stage(v, ix, d, dirmask, lane):
        W = v.shape[1]
        lowbit = (lane & d) == 0
        vp = jnp.where(lowbit, pltpu.roll(v, (-d) % W, 1), pltpu.roll(v, d, 1))
        ip = jnp.where(lowbit, pltpu.roll(ix, (-d) % W, 1), pltpu.roll(ix, d, 1))
        mine_wins = (v > vp) | ((v == vp) & (ix < ip))
        keep = mine_wins == (lowbit == dirmask)
        return jnp.where(keep, v, vp), jnp.where(keep, ix, ip)

    for bs_log in range(1, 7):
        dirmask = (lane_g & (1 << bs_log)) == 0
        for d_log in reversed(range(bs_log)):
            v, ix = stage(v, ix, 1 << d_log, dirmask, lane_g)
            if pump is not None:
                pump()
    while True:
        W = v.shape[1]
        lane = lane_g[:, :W]
        v, ix = stage(v, ix, 64, lane >= 0, lane)
        dirmask = (lane & 128) == 0
        for d_log in reversed(range(6)):
            v, ix = stage(v, ix, 1 << d_log, dirmask, lane)
            if pump is not None:
                pump()
        if W == 128:
            break
        v = jnp.concatenate([v[:, m * 128:m * 128 + 64]
                             for m in range(W // 128)], axis=1)
        ix = jnp.concatenate([ix[:, m * 128:m * 128 + 64]
                              for m in range(W // 128)], axis=1)
    return v[:, :_K], ix[:, :_K]


def _gather_row(i, prev, idx_s, tv_s, emb_ref, G, reps):
    iota_l = jax.lax.broadcasted_iota(jnp.int32, (_K, G), 1)
    oh_t = (iota_l == idx_s[:, i:i + 1]).astype(jnp.bfloat16)
    g = jax.lax.dot_general(oh_t, prev[i],
                            (((1,), (0,)), ((), ())),
                            preferred_element_type=jnp.float32)
    g_t = jnp.transpose(g) * tv_s[i:i + 1, :]
    emb_ref[i] = jnp.concatenate([g_t] * reps, axis=1)


def _pipe_body(ne_hbm, hs_ref, w_ref, b_ref,
               emb_ref, pol_ref, scr_ref, ent_ref, idx_ref,
               buf, idx_s, tv_s, sem):
    f32 = jnp.float32
    nblk = pl.num_programs(0) - 1
    _, G, D = emb_ref.shape[0], ne_hbm.shape[1], ne_hbm.shape[2]
    reps = D // _K
    t = pl.program_id(0)
    prev = buf.at[(t - 1) % 2]

    @pl.when(t == 0)
    def _():
        pltpu.make_async_copy(ne_hbm.at[pl.ds(0, _BBLK)],
                              buf.at[0], sem.at[0]).start()

    @pl.when(t == nblk)
    def _():
        for i in range(_BBLK):
            _gather_row(i, prev, idx_s, tv_s, emb_ref, G, reps)

    @pl.when(t < nblk)
    def _():
        pltpu.make_async_copy(ne_hbm.at[pl.ds(t * _BBLK, _BBLK)],
                              buf.at[t % 2], sem.at[t % 2]).wait()
        cur = buf.at[t % 2]

        hs = hs_ref[...]
        W_m = w_ref[...]
        bb = b_ref[...]
        scorer = jnp.tanh(
            jax.lax.dot_general(hs, W_m, (((1,), (1,)), ((), ())),
                                preferred_element_type=f32) + bb)
        scr_ref[...] = scorer
        norm = jnp.sqrt(jnp.sum(scorer * scorer, axis=1, keepdims=True))

        rows = []
        for i in range(_BBLK):
            s_i = jax.lax.dot_general(scorer[i:i + 1], cur[i],
                                      (((1,), (1,)), ((), ())),
                                      preferred_element_type=f32)
            rows.append(s_i)
        scores = jnp.concatenate(rows, axis=0) / norm

        state = {'stages': 0, 'rows': 0, 'prefetched': False}

        def pump():
            state['stages'] += 1
            if state['stages'] % 3 == 0 and state['rows'] < _BBLK:
                _gather_row(state['rows'], prev, idx_s, tv_s,
                            emb_ref, G, reps)
                state['rows'] += 1
            elif state['rows'] >= _BBLK and not state['prefetched']:
                state['prefetched'] = True

                @pl.when(t + 1 < nblk)
                def _():
                    nxt = (t + 1) % 2
                    pltpu.make_async_copy(
                        ne_hbm.at[pl.ds((t + 1) * _BBLK, _BBLK)],
                        buf.at[nxt], sem.at[nxt]).start()

        vals, idxs = _topk_bitonic(scores, pump)

        m0 = vals[:, 0:1]
        e = jnp.exp(scores - m0)
        z = jnp.sum(e, axis=1, keepdims=True)
        logz = m0 + jnp.log(z)
        ps = jnp.sum(e * scores, axis=1, keepdims=True) / z
        ent_ref[...] = logz - ps
        pol_ref[...] = jnp.mean(vals, axis=1, keepdims=True) - logz
        idx_ref[...] = jnp.concatenate([idxs] * reps, axis=1)

        idx_s[...] = jnp.transpose(idxs)
        tv_s[...] = jnp.tanh(vals)


def kernel(node_embs, mask, h_selector, W, b):
    del mask
    B, G, D = node_embs.shape
    RNN = h_selector.shape[1]
    b2 = b.reshape(1, D)
    nblk = B // _BBLK
    last = nblk - 1

    out_shape = (
        jax.ShapeDtypeStruct((B, D, D), jnp.float32),
        jax.ShapeDtypeStruct((B, 1), jnp.float32),
        jax.ShapeDtypeStruct((B, D), jnp.float32),
        jax.ShapeDtypeStruct((B, 1), jnp.float32),
        jax.ShapeDtypeStruct((B, D), jnp.int32),
    )
    emb, pol, scr, ent, idx = pl.pallas_call(
        _pipe_body,
        grid=(nblk + 1,),
        in_specs=[
            pl.BlockSpec(memory_space=pl.ANY),
            pl.BlockSpec((_BBLK, RNN), lambda t: (jnp.minimum(t, last), 0)),
            pl.BlockSpec((D, RNN), lambda t: (0, 0)),
            pl.BlockSpec((1, D), lambda t: (0, 0)),
        ],
        out_specs=[
            pl.BlockSpec((_BBLK, D, D),
                         lambda t: (jnp.maximum(t - 1, 0), 0, 0)),
            pl.BlockSpec((_BBLK, 1), lambda t: (jnp.minimum(t, last), 0)),
            pl.BlockSpec((_BBLK, D), lambda t: (jnp.minimum(t, last), 0)),
            pl.BlockSpec((_BBLK, 1), lambda t: (jnp.minimum(t, last), 0)),
            pl.BlockSpec((_BBLK, D), lambda t: (jnp.minimum(t, last), 0)),
        ],
        out_shape=out_shape,
        scratch_shapes=[
            pltpu.VMEM((2, _BBLK, G, D), jnp.float32),
            pltpu.VMEM((_K, _BBLK), jnp.int32),
            pltpu.VMEM((_BBLK, _K), jnp.float32),
            pltpu.SemaphoreType.DMA((2,)),
        ],
    )(node_embs, h_selector, W, b2)
    return emb, pol[:, 0], scr, ent[:, 0], idx


# speedup vs baseline: 1.3575x; 1.3575x over previous
import jax
import jax.numpy as jnp
from jax.experimental import pallas as pl
from jax.experimental.pallas import tpu as pltpu

_BBLK = 8
_K = 64


def _topk_bitonic(scores):
    R, G = scores.shape
    lane_g = jax.lax.broadcasted_iota(jnp.int32, (R, G), 1)
    v = scores
    ix = lane_g

    def stage(v, ix, d, dirmask, lane):
        W = v.shape[1]
        lowbit = (lane & d) == 0
        vp = jnp.where(lowbit, pltpu.roll(v, (-d) % W, 1), pltpu.roll(v, d, 1))
        ip = jnp.where(lowbit, pltpu.roll(ix, (-d) % W, 1), pltpu.roll(ix, d, 1))
        mine_wins = (v > vp) | ((v == vp) & (ix < ip))
        keep = mine_wins == (lowbit == dirmask)
        return jnp.where(keep, v, vp), jnp.where(keep, ix, ip)

    for bs_log in range(1, 7):
        dirmask = (lane_g & (1 << bs_log)) == 0
        for d_log in reversed(range(bs_log)):
            v, ix = stage(v, ix, 1 << d_log, dirmask, lane_g)
    while True:
        W = v.shape[1]
        lane = lane_g[:, :W]
        v, ix = stage(v, ix, 64, lane >= 0, lane)
        dirmask = (lane & 128) == 0
        for d_log in reversed(range(6)):
            v, ix = stage(v, ix, 1 << d_log, dirmask, lane)
        if W == 128:
            break
        v = jnp.concatenate([v[:, m * 128:m * 128 + 64]
                             for m in range(W // 128)], axis=1)
        ix = jnp.concatenate([ix[:, m * 128:m * 128 + 64]
                              for m in range(W // 128)], axis=1)
    return v[:, :_K], ix[:, :_K]


def _fused_body(ne_ref, hs_ref, w_ref, b_ref,
                emb_ref, pol_ref, scr_ref, ent_ref, idx_ref):
    f32 = jnp.float32
    hs = hs_ref[...]
    W = w_ref[...]
    bb = b_ref[...]

    scorer = jnp.tanh(
        jax.lax.dot_general(hs, W, (((1,), (1,)), ((), ())),
                            preferred_element_type=f32) + bb)
    scr_ref[...] = scorer
    norm = jnp.sqrt(jnp.sum(scorer * scorer, axis=1, keepdims=True))

    rows = []
    for i in range(_BBLK):
        s_i = jax.lax.dot_general(scorer[i:i + 1], ne_ref[i],
                                  (((1,), (1,)), ((), ())),
                                  preferred_element_type=f32)
        rows.append(s_i)
    scores = jnp.concatenate(rows, axis=0) / norm

    G = scores.shape[1]
    vals, idxs = _topk_bitonic(scores)

    m0 = vals[:, 0:1]
    e = jnp.exp(scores - m0)
    z = jnp.sum(e, axis=1, keepdims=True)
    logz = m0 + jnp.log(z)
    ps = jnp.sum(e * scores, axis=1, keepdims=True) / z
    ent_ref[...] = logz - ps
    pol_ref[...] = jnp.mean(vals, axis=1, keepdims=True) - logz

    reps = idx_ref.shape[1] // _K
    idx_ref[...] = jnp.concatenate([idxs] * reps, axis=1)

    tanh_vals = jnp.tanh(vals)
    idxs_t = jnp.transpose(idxs)
    iota_l = jax.lax.broadcasted_iota(jnp.int32, (_K, G), 1)
    for i in range(_BBLK):
        oh_t = (iota_l == idxs_t[:, i:i + 1]).astype(jnp.bfloat16)
        g = jax.lax.dot_general(oh_t, ne_ref[i],
                                (((1,), (0,)), ((), ())),
                                preferred_element_type=f32)
        g_t = jnp.transpose(g) * tanh_vals[i:i + 1, :]
        emb_ref[i] = jnp.concatenate([g_t] * reps, axis=1)


def kernel(node_embs, mask, h_selector, W, b):
    del mask
    B, G, D = node_embs.shape
    RNN = h_selector.shape[1]
    b2 = b.reshape(1, D)
    nblk = B // _BBLK

    out_shape = (
        jax.ShapeDtypeStruct((B, D, D), jnp.float32),
        jax.ShapeDtypeStruct((B, 1), jnp.float32),
        jax.ShapeDtypeStruct((B, D), jnp.float32),
        jax.ShapeDtypeStruct((B, 1), jnp.float32),
        jax.ShapeDtypeStruct((B, D), jnp.int32),
    )
    emb, pol, scr, ent, idx = pl.pallas_call(
        _fused_body,
        grid=(nblk,),
        in_specs=[
            pl.BlockSpec((_BBLK, G, D), lambda i: (i, 0, 0)),
            pl.BlockSpec((_BBLK, RNN), lambda i: (i, 0)),
            pl.BlockSpec((D, RNN), lambda i: (0, 0)),
            pl.BlockSpec((1, D), lambda i: (0, 0)),
        ],
        out_specs=[
            pl.BlockSpec((_BBLK, D, D), lambda i: (i, 0, 0)),
            pl.BlockSpec((_BBLK, 1), lambda i: (i, 0)),
            pl.BlockSpec((_BBLK, D), lambda i: (i, 0)),
            pl.BlockSpec((_BBLK, 1), lambda i: (i, 0)),
            pl.BlockSpec((_BBLK, D), lambda i: (i, 0)),
        ],
        out_shape=out_shape,
    )(node_embs, h_selector, W, b2)
    return emb, pol[:, 0], scr, ent[:, 0], idx
